# baseline (speedup 1.0000x reference)
import jax
import jax.numpy as jnp
from jax.experimental import pallas as pl


def _scale_body(logits_ref, temp_ref, out_ref):
    inv = 1.0 / jax.nn.sigmoid(temp_ref[...])
    out_ref[...] = logits_ref[...] * inv


def kernel(logits, temp):
    B, V = logits.shape
    BV = 57344
    grid = (pl.cdiv(V, BV),)
    return pl.pallas_call(
        _scale_body,
        grid=grid,
        in_specs=[
            pl.BlockSpec((B, BV), lambda i: (0, i)),
            pl.BlockSpec((B, 1), lambda i: (0, 0)),
        ],
        out_specs=pl.BlockSpec((B, BV), lambda i: (0, i)),
        out_shape=jax.ShapeDtypeStruct((B, V), logits.dtype),
    )(logits, temp)


# speedup vs baseline: 1.0003x; 1.0003x over previous
import jax
import jax.numpy as jnp
from jax.experimental import pallas as pl
from jax.experimental.pallas import tpu as pltpu


def _scale_body(logits_ref, temp_ref, out_ref):
    inv = 1.0 / jax.nn.sigmoid(temp_ref[...])
    out_ref[...] = logits_ref[...] * inv


def kernel(logits, temp):
    B, V = logits.shape
    BV = 57344
    grid = (pl.cdiv(V, BV),)
    return pl.pallas_call(
        _scale_body,
        grid=grid,
        in_specs=[
            pl.BlockSpec((B, BV), lambda i: (0, i)),
            pl.BlockSpec((B, 1), lambda i: (0, 0)),
        ],
        out_specs=pl.BlockSpec((B, BV), lambda i: (0, i)),
        out_shape=jax.ShapeDtypeStruct((B, V), logits.dtype),
        compiler_params=pltpu.CompilerParams(dimension_semantics=("parallel",)),
    )(logits, temp)
